# baseline (speedup 1.0000x reference)
import jax
import jax.numpy as jnp
from jax.experimental import pallas as pl

N = 256
P_BLK = 512


def _body(idx_ref, feat_ref, out_ref):
    blk = pl.program_id(1)
    base = blk * P_BLK
    ii = idx_ref[0, 0, :]
    jj = idx_ref[0, 1, :]
    p = ii * N + jj
    rows = jax.lax.broadcasted_iota(jnp.int32, (P_BLK, p.shape[0]), 0) + base
    onehot = (rows == p[None, :]).astype(jnp.float32)
    out_ref[0, :, :] = jnp.dot(onehot, feat_ref[0],
                               preferred_element_type=jnp.float32)


def kernel(edge_features_batch, pair_indices_batch):
    B, E, F = edge_features_batch.shape
    P = N * N
    idx = pair_indices_batch.astype(jnp.int32).transpose(0, 2, 1)
    out = pl.pallas_call(
        _body,
        grid=(B, P // P_BLK),
        in_specs=[
            pl.BlockSpec((1, 2, E), lambda b, k: (b, 0, 0)),
            pl.BlockSpec((1, E, F), lambda b, k: (b, 0, 0)),
        ],
        out_specs=pl.BlockSpec((1, P_BLK, F), lambda b, k: (b, k, 0)),
        out_shape=jax.ShapeDtypeStruct((B, P, F), jnp.float32),
    )(idx, edge_features_batch)
    return out.reshape(B, N, N, F)


# speedup vs baseline: 2.0480x; 2.0480x over previous
import functools

import jax
import jax.numpy as jnp
from jax import lax
from jax.experimental import pallas as pl
from jax.experimental.pallas import tpu as pltpu
from jax.experimental.pallas import tpu_sc as plsc

N = 256
L = 16

CHUNK = 4096
ZROWS = CHUNK // 16


def _sc_body(nedges, rows_total, idx_hbm, feat_hbm, zeros_hbm, out_hbm,
             idx_v, g_v, cdst_v, csrc_v, grp_rows, zero_v,
             buf0, buf1, sem0, sem1):
    c = lax.axis_index("c")
    s = lax.axis_index("s")
    epw = nedges // 16
    base_e = s * epw
    half = rows_total // 2
    sc_base = c * half
    nch = half // CHUNK

    pltpu.sync_copy(idx_hbm.at[:, pl.ds(base_e, epw)], idx_v)
    pltpu.sync_copy(zeros_hbm, zero_v)

    def g_step(t, _):
        iv = idx_v[0, pl.ds(t * L, L)]
        jv = idx_v[1, pl.ds(t * L, L)]
        b = (base_e + t * L) // (nedges // 4)
        g_v[pl.ds(t * L, L)] = (b * (N * N) + iv * N + jv).astype(jnp.int32)
        return ()

    lax.fori_loop(0, epw // L, g_step, (), unroll=False)

    def do_chunk(ch, buf, sem):
        lo = sc_base + ch * CHUNK
        my0 = s * ZROWS

        @pl.when(ch >= 2)
        def _():
            pltpu.make_async_copy(
                buf.at[pl.ds(my0, ZROWS)],
                out_hbm.at[pl.ds(lo + my0, ZROWS)], sem).wait()

        pltpu.sync_copy(zero_v, buf.at[pl.ds(my0, ZROWS)])
        plsc.subcore_barrier()

        def comp_step(t, k):
            gv = g_v[pl.ds(t * L, L)]
            rel = gv - lo
            m = (rel >= 0) & (rel < CHUNK)
            cum = plsc.cumsum(m.astype(jnp.int32))
            pos = k + cum - 1
            plsc.store_scatter(cdst_v, [pos], rel, mask=m)
            src = base_e + t * L + lax.iota(jnp.int32, L)
            plsc.store_scatter(csrc_v, [pos], src, mask=m)
            return k + cum[L - 1]

        k = lax.fori_loop(0, epw // L, comp_step, jnp.int32(0), unroll=False)

        cdst_v[pl.ds(k, L)] = jnp.full((L,), CHUNK, jnp.int32)
        csrc_v[pl.ds(k, L)] = jnp.full((L,), base_e, jnp.int32)

        def grp_step(gi, _):
            dv = cdst_v[pl.ds(gi * L, L)]
            sv = csrc_v[pl.ds(gi * L, L)]
            pltpu.sync_copy(feat_hbm.at[sv], grp_rows)
            pltpu.sync_copy(grp_rows, buf.at[dv], add=True)
            return ()

        lax.fori_loop(0, (k + L - 1) // L, grp_step, (), unroll=False)
        plsc.subcore_barrier()

        pltpu.make_async_copy(
            buf.at[pl.ds(my0, ZROWS)],
            out_hbm.at[pl.ds(lo + my0, ZROWS)], sem).start()

    def pair(cc, _):
        do_chunk(2 * cc, buf0, sem0)
        do_chunk(2 * cc + 1, buf1, sem1)
        return ()

    lax.fori_loop(0, nch // 2, pair, (), unroll=False)

    my0 = s * ZROWS
    pltpu.make_async_copy(buf0.at[pl.ds(my0, ZROWS)],
                          out_hbm.at[pl.ds(my0, ZROWS)], sem0).wait()
    pltpu.make_async_copy(buf1.at[pl.ds(my0, ZROWS)],
                          out_hbm.at[pl.ds(my0, ZROWS)], sem1).wait()


def kernel(edge_features_batch, pair_indices_batch):
    B, E, F = edge_features_batch.shape
    P = N * N
    rows_total = B * P
    nedges = B * E

    feat = edge_features_batch.reshape(nedges, F)
    idx = pair_indices_batch.astype(jnp.int32).reshape(nedges, 2).T
    zeros = jnp.zeros((ZROWS, F), jnp.float32)
    epw = nedges // 16

    mesh = plsc.VectorSubcoreMesh(core_axis_name="c", subcore_axis_name="s")
    run = pl.kernel(
        functools.partial(_sc_body, nedges, rows_total),
        mesh=mesh,
        compiler_params=pltpu.CompilerParams(needs_layout_passes=False),
        out_type=jax.ShapeDtypeStruct((rows_total, F), jnp.float32),
        scratch_types=[
            pltpu.VMEM((2, epw), jnp.int32),
            pltpu.VMEM((epw,), jnp.int32),
            pltpu.VMEM((epw + 2 * L,), jnp.int32),
            pltpu.VMEM((epw + 2 * L,), jnp.int32),
            pltpu.VMEM((L, F), jnp.float32),
            pltpu.VMEM((ZROWS, F), jnp.float32),
            pltpu.VMEM_SHARED((CHUNK + L, F), jnp.float32),
            pltpu.VMEM_SHARED((CHUNK + L, F), jnp.float32),
            pltpu.SemaphoreType.DMA,
            pltpu.SemaphoreType.DMA,
        ],
    )
    out = run(idx, feat, zeros)
    return out.reshape(B, N, N, F)


# speedup vs baseline: 2.2515x; 1.0993x over previous
import functools

import jax
import jax.numpy as jnp
from jax import lax
from jax.experimental import pallas as pl
from jax.experimental.pallas import tpu as pltpu
from jax.experimental.pallas import tpu_sc as plsc

N = 256
L = 16

CHUNK = 8192
ZINIT = 64
WAVE = 16


def _sc_body(nedges, rows_total, idx_hbm, feat_hbm, zeros_hbm, out_hbm,
             idx_v, g_v, cdst0, csrc0, cdst1, csrc1, stage_v, zero_v,
             buf, wsem, gsem):
    c = lax.axis_index("c")
    s = lax.axis_index("s")
    epw = nedges // 16
    base_e = s * epw
    half = rows_total // 2
    sc_base = c * half
    nch = half // CHUNK
    zrows = CHUNK // 16
    my0 = s * zrows

    pltpu.sync_copy(idx_hbm.at[:, pl.ds(base_e, epw)], idx_v)
    pltpu.sync_copy(zeros_hbm, zero_v)
    for z in range(zrows // ZINIT):
        pltpu.sync_copy(zero_v, buf.at[pl.ds(my0 + z * ZINIT, ZINIT)])

    def g_step(t, _):
        iv = idx_v[0, pl.ds(t * L, L)]
        jv = idx_v[1, pl.ds(t * L, L)]
        b = (base_e + t * L) // (nedges // 4)
        g_v[pl.ds(t * L, L)] = b * (N * N) + iv * N + jv
        return ()

    lax.fori_loop(0, epw // L, g_step, (), unroll=False)
    plsc.subcore_barrier()

    def compact(lo, cdst, csrc):
        def comp_step(t, k):
            gv = g_v[pl.ds(t * L, L)]
            rel = gv - lo
            m = (rel >= 0) & (rel < CHUNK)
            cum = plsc.cumsum(m.astype(jnp.int32))
            pos = k + cum - 1
            plsc.store_scatter(cdst, [pos], rel, mask=m)
            src = base_e + t * L + lax.iota(jnp.int32, L)
            plsc.store_scatter(csrc, [pos], src, mask=m)
            return k + cum[L - 1]

        k = lax.fori_loop(0, epw // L, comp_step, jnp.int32(0), unroll=False)
        cdst[pl.ds(k, L)] = jnp.full((L,), CHUNK, jnp.int32)
        csrc[pl.ds(k, L)] = jnp.full((L,), base_e, jnp.int32)
        return k

    def wo_desc(ch):
        return pltpu.make_async_copy(
            buf.at[pl.ds(my0, zrows)],
            out_hbm.at[pl.ds(sc_base + ch * CHUNK + my0, zrows)], wsem)

    def gather_desc():
        return pltpu.make_async_copy(
            feat_hbm.at[csrc0[pl.ds(0, L)]], stage_v.at[pl.ds(0, L)], gsem)

    def add_desc():
        return pltpu.make_async_copy(
            stage_v.at[pl.ds(0, L)], buf.at[cdst0[pl.ds(0, L)]], gsem)

    def do_chunk(ch, cdst, csrc, cdst_prev, csrc_prev, k_prev):
        lo = sc_base + ch * CHUNK
        k = compact(lo, cdst, csrc)
        ng = (k + L - 1) // L

        @pl.when(ch >= 1)
        def _():
            wo_desc(ch - 1).wait()

        plsc.subcore_barrier()

        @pl.when(ch >= 1)
        def _():
            def rz_step(gi, _):
                dv = cdst_prev[pl.ds(gi * L, L)]
                pltpu.sync_copy(zero_v.at[pl.ds(0, L)], buf.at[dv])
                return ()

            lax.fori_loop(0, (k_prev + L - 1) // L, rz_step, (), unroll=False)

        plsc.subcore_barrier()

        def wave_step(w, _):
            g0 = w * WAVE
            gcnt = jnp.minimum(ng - g0, WAVE)

            def gather_step(q, _):
                sv = csrc[pl.ds((g0 + q) * L, L)]
                pltpu.async_copy(feat_hbm.at[sv],
                                 stage_v.at[pl.ds(q * L, L)], gsem)
                return ()

            lax.fori_loop(0, gcnt, gather_step, (), unroll=False)
            lax.fori_loop(0, gcnt, lambda q, _: (gather_desc().wait(),)[1:],
                          (), unroll=False)

            def add_step(q, _):
                dv = cdst[pl.ds((g0 + q) * L, L)]
                pltpu.async_copy(stage_v.at[pl.ds(q * L, L)],
                                 buf.at[dv], gsem, add=True)
                return ()

            lax.fori_loop(0, gcnt, add_step, (), unroll=False)
            lax.fori_loop(0, gcnt, lambda q, _: (add_desc().wait(),)[1:],
                          (), unroll=False)
            return ()

        lax.fori_loop(0, (ng + WAVE - 1) // WAVE, wave_step, (),
                      unroll=False)
        plsc.subcore_barrier()

        wo_desc(ch).start()
        return k

    def pair(cc, ks):
        k0, k1 = ks
        k0 = do_chunk(2 * cc, cdst0, csrc0, cdst1, csrc1, k1)
        k1 = do_chunk(2 * cc + 1, cdst1, csrc1, cdst0, csrc0, k0)
        return (k0, k1)

    lax.fori_loop(0, nch // 2, pair, (jnp.int32(0), jnp.int32(0)),
                  unroll=False)
    wo_desc(nch - 1).wait()


def kernel(edge_features_batch, pair_indices_batch):
    B, E, F = edge_features_batch.shape
    P = N * N
    rows_total = B * P
    nedges = B * E

    feat = edge_features_batch.reshape(nedges, F)
    idx = pair_indices_batch.astype(jnp.int32).reshape(nedges, 2).T
    zeros = jnp.zeros((ZINIT, F), jnp.float32)
    epw = nedges // 16

    mesh = plsc.VectorSubcoreMesh(core_axis_name="c", subcore_axis_name="s")
    run = pl.kernel(
        functools.partial(_sc_body, nedges, rows_total),
        mesh=mesh,
        compiler_params=pltpu.CompilerParams(needs_layout_passes=False),
        out_type=jax.ShapeDtypeStruct((rows_total, F), jnp.float32),
        scratch_types=[
            pltpu.VMEM((2, epw), jnp.int32),
            pltpu.VMEM((epw,), jnp.int32),
            pltpu.VMEM((epw + 2 * L,), jnp.int32),
            pltpu.VMEM((epw + 2 * L,), jnp.int32),
            pltpu.VMEM((epw + 2 * L,), jnp.int32),
            pltpu.VMEM((epw + 2 * L,), jnp.int32),
            pltpu.VMEM((WAVE * L, F), jnp.float32),
            pltpu.VMEM((ZINIT, F), jnp.float32),
            pltpu.VMEM_SHARED((CHUNK + L, F), jnp.float32),
            pltpu.SemaphoreType.DMA,
            pltpu.SemaphoreType.DMA,
        ],
    )
    out = run(idx, feat, zeros)
    return out.reshape(B, N, N, F)


# speedup vs baseline: 2.8034x; 1.2452x over previous
import functools

import jax
import jax.numpy as jnp
from jax import lax
from jax.experimental import pallas as pl
from jax.experimental.pallas import tpu as pltpu
from jax.experimental.pallas import tpu_sc as plsc

N = 256
L = 16

CHUNK = 8192
ZINIT = 64
WAVE = 16


def _sc_body(nedges, rows_total, idx_hbm, feat_hbm, zeros_hbm, out_hbm,
             idx_v, g_v, cdst0, csrc0, cdst1, csrc1, stage_v, zero_v,
             buf, wsem, gsem, zsem):
    c = lax.axis_index("c")
    s = lax.axis_index("s")
    epw = nedges // 16
    base_e = s * epw
    half = rows_total // 2
    sc_base = c * half
    nch = half // CHUNK
    zrows = CHUNK // 16
    my0 = s * zrows

    pltpu.sync_copy(idx_hbm.at[:, pl.ds(base_e, epw)], idx_v)
    pltpu.sync_copy(zeros_hbm, zero_v)
    for z in range(zrows // ZINIT):
        pltpu.sync_copy(zero_v, buf.at[pl.ds(my0 + z * ZINIT, ZINIT)])

    def g_step(t, _):
        iv = idx_v[0, pl.ds(t * L, L)]
        jv = idx_v[1, pl.ds(t * L, L)]
        b = (base_e + t * L) // (nedges // 4)
        g_v[pl.ds(t * L, L)] = b * (N * N) + iv * N + jv
        return ()

    lax.fori_loop(0, epw // L, g_step, (), unroll=False)
    plsc.subcore_barrier()

    def compact(lo, cdst, csrc):
        def comp_step(t, k):
            gv = g_v[pl.ds(t * L, L)]
            rel = gv - lo
            m = (rel >= 0) & (rel < CHUNK)
            cum = plsc.cumsum(m.astype(jnp.int32))
            pos = k + cum - 1
            plsc.store_scatter(cdst, [pos], rel, mask=m)
            src = base_e + t * L + lax.iota(jnp.int32, L)
            plsc.store_scatter(csrc, [pos], src, mask=m)
            return k + cum[L - 1]

        k = lax.fori_loop(0, epw // L, comp_step, jnp.int32(0), unroll=False)
        cdst[pl.ds(k, L)] = jnp.full((L,), CHUNK, jnp.int32)
        csrc[pl.ds(k, L)] = jnp.full((L,), base_e, jnp.int32)
        return k

    def wo_desc(ch):
        return pltpu.make_async_copy(
            buf.at[pl.ds(my0, zrows)],
            out_hbm.at[pl.ds(sc_base + ch * CHUNK + my0, zrows)], wsem)

    def gather_desc():
        return pltpu.make_async_copy(
            feat_hbm.at[csrc0[pl.ds(0, L)]], stage_v.at[pl.ds(0, L)], gsem)

    def add_desc():
        return pltpu.make_async_copy(
            stage_v.at[pl.ds(0, L)], buf.at[cdst0[pl.ds(0, L)]], gsem)

    def rz_desc():
        return pltpu.make_async_copy(
            zero_v.at[pl.ds(0, L)], buf.at[cdst0[pl.ds(0, L)]], zsem)

    def fire_gathers(csrc, g0, gcnt, sem):
        def gather_step(q, _):
            sv = csrc[pl.ds((g0 + q) * L, L)]
            pltpu.async_copy(feat_hbm.at[sv],
                             stage_v.at[pl.ds(q * L, L)], sem)
            return ()

        lax.fori_loop(0, gcnt, gather_step, (), unroll=False)

    def fire_adds(cdst, g0, gcnt, sem):
        def add_step(q, _):
            dv = cdst[pl.ds((g0 + q) * L, L)]
            pltpu.async_copy(stage_v.at[pl.ds(q * L, L)],
                             buf.at[dv], sem, add=True)
            return ()

        lax.fori_loop(0, gcnt, add_step, (), unroll=False)

    def do_chunk(ch, cdst, csrc, cdst_prev, csrc_prev, k_prev):
        lo = sc_base + ch * CHUNK
        k = compact(lo, cdst, csrc)
        ng = (k + L - 1) // L
        gcnt0 = jnp.minimum(ng, WAVE)
        fire_gathers(csrc, 0, gcnt0, gsem)

        @pl.when(ch >= 1)
        def _():
            wo_desc(ch - 1).wait()

        plsc.subcore_barrier()

        @pl.when(ch >= 1)
        def _():
            ngp = (k_prev + L - 1) // L

            def rz_step(gi, _):
                dv = cdst_prev[pl.ds(gi * L, L)]
                pltpu.async_copy(zero_v.at[pl.ds(0, L)], buf.at[dv], zsem)
                return ()

            lax.fori_loop(0, ngp, rz_step, (), unroll=False)
            lax.fori_loop(0, ngp, lambda gi, _: (rz_desc().wait(),)[1:],
                          (), unroll=False)

        plsc.subcore_barrier()

        lax.fori_loop(0, gcnt0, lambda q, _: (gather_desc().wait(),)[1:],
                      (), unroll=False)
        fire_adds(cdst, 0, gcnt0, gsem)
        lax.fori_loop(0, gcnt0, lambda q, _: (add_desc().wait(),)[1:],
                      (), unroll=False)

        @pl.when(ng > WAVE)
        def _():
            def wave_step(w, _):
                g0 = w * WAVE
                gcnt = jnp.minimum(ng - g0, WAVE)
                fire_gathers(csrc, g0, gcnt, gsem)
                lax.fori_loop(0, gcnt,
                              lambda q, _: (gather_desc().wait(),)[1:],
                              (), unroll=False)
                fire_adds(cdst, g0, gcnt, gsem)
                lax.fori_loop(0, gcnt,
                              lambda q, _: (add_desc().wait(),)[1:],
                              (), unroll=False)
                return ()

            lax.fori_loop(1, (ng + WAVE - 1) // WAVE, wave_step, (),
                          unroll=False)

        plsc.subcore_barrier()

        wo_desc(ch).start()
        return k

    def pair(cc, ks):
        k0, k1 = ks
        k0 = do_chunk(2 * cc, cdst0, csrc0, cdst1, csrc1, k1)
        k1 = do_chunk(2 * cc + 1, cdst1, csrc1, cdst0, csrc0, k0)
        return (k0, k1)

    lax.fori_loop(0, nch // 2, pair, (jnp.int32(0), jnp.int32(0)),
                  unroll=False)
    wo_desc(nch - 1).wait()


def kernel(edge_features_batch, pair_indices_batch):
    B, E, F = edge_features_batch.shape
    P = N * N
    rows_total = B * P
    nedges = B * E

    feat = edge_features_batch.reshape(nedges, F)
    idx = pair_indices_batch.astype(jnp.int32).reshape(nedges, 2).T
    zeros = jnp.zeros((ZINIT, F), jnp.float32)
    epw = nedges // 16

    mesh = plsc.VectorSubcoreMesh(core_axis_name="c", subcore_axis_name="s")
    run = pl.kernel(
        functools.partial(_sc_body, nedges, rows_total),
        mesh=mesh,
        compiler_params=pltpu.CompilerParams(needs_layout_passes=False),
        out_type=jax.ShapeDtypeStruct((rows_total, F), jnp.float32),
        scratch_types=[
            pltpu.VMEM((2, epw), jnp.int32),
            pltpu.VMEM((epw,), jnp.int32),
            pltpu.VMEM((epw + 2 * L,), jnp.int32),
            pltpu.VMEM((epw + 2 * L,), jnp.int32),
            pltpu.VMEM((epw + 2 * L,), jnp.int32),
            pltpu.VMEM((epw + 2 * L,), jnp.int32),
            pltpu.VMEM((WAVE * L, F), jnp.float32),
            pltpu.VMEM((ZINIT, F), jnp.float32),
            pltpu.VMEM_SHARED((CHUNK + L, F), jnp.float32),
            pltpu.SemaphoreType.DMA,
            pltpu.SemaphoreType.DMA,
            pltpu.SemaphoreType.DMA,
        ],
    )
    out = run(idx, feat, zeros)
    return out.reshape(B, N, N, F)
